# baseline (speedup 1.0000x reference)
import functools

import jax
import jax.numpy as jnp
from jax.experimental import pallas as pl
from jax.experimental.pallas import tpu as pltpu


def _body(x_ref, j_ref, h_ref, o_ref, wt, *, Q, L, BI, LQ, BJ, NS):
    t = pl.program_id(0)
    s = pl.program_id(1)
    smax = (BI * t + BI - 1) // BJ

    @pl.when(jnp.logical_and(t == 0, s == 0))
    def _():
        wt[...] = jnp.zeros_like(wt)

    @pl.when(s <= smax)
    def _():
        row_j = s * BJ + jax.lax.broadcasted_iota(jnp.int32, (BJ, 1), 0)
        for il in range(BI):
            jb = j_ref[il]
            keep = row_j < (t * BI + il)
            for u in range(Q):
                piece = jnp.where(keep, jb[:, u * Q:(u + 1) * Q], 0.0)
                wt[pl.ds(u * L + s * BJ, BJ), il * Q:(il + 1) * Q] = (
                    piece.astype(jnp.bfloat16))

    @pl.when(s == NS - 1)
    def _():
        acc = jnp.dot(x_ref[...], wt[...], preferred_element_type=jnp.float32)
        o_ref[0] = acc + h_ref[0]


def kernel(X_oh, h_pos, J):
    M, L, Q = X_oh.shape
    LQ = L * Q
    BI = 8
    BJ = 32
    NS = L // BJ
    TN = BI * Q
    n_col = L // BI

    J4 = J.reshape(L, L, Q * Q)
    Xp = X_oh.astype(jnp.bfloat16).transpose(0, 2, 1).reshape(M, LQ)
    hf = h_pos.reshape(n_col, 1, TN)

    out = pl.pallas_call(
        functools.partial(_body, Q=Q, L=L, BI=BI, LQ=LQ, BJ=BJ, NS=NS),
        grid=(n_col, NS),
        in_specs=[
            pl.BlockSpec((M, LQ), lambda t, s: (0, 0)),
            pl.BlockSpec(
                (BI, BJ, Q * Q),
                lambda t, s: (t, jnp.minimum(s, (BI * t + BI - 1) // BJ), 0)),
            pl.BlockSpec((1, 1, TN), lambda t, s: (t, 0, 0)),
        ],
        out_specs=pl.BlockSpec((1, M, TN), lambda t, s: (t, 0, 0)),
        out_shape=jax.ShapeDtypeStruct((n_col, M, TN), jnp.float32),
        scratch_shapes=[
            pltpu.VMEM((LQ, TN), jnp.bfloat16),
        ],
    )(Xp, J4, hf)
    return out.transpose(1, 0, 2).reshape(M, L, Q)
